# baseline (speedup 1.0000x reference)
import functools

import jax
import jax.numpy as jnp
from jax import lax
from jax.experimental import pallas as pl
from jax.experimental.pallas import tpu as pltpu
from jax.experimental.pallas import tpu_sc as plsc

E = 8
D = 2048
N = 2048
TM = 128
TN = 1024
TOT = N * 2 + E * TM
NT = TOT // TM

NC = 2
NS = 16
NW = NC * NS

GCH = 8
G_PER_W = TOT // NW
GN = G_PER_W // GCH
CCH = 8
C_PER_W = N // NW
CN = C_PER_W // CCH


TG = 256


def _gate_body(x_ref, wgt_ref, bg_ref,
               s128_ref, e0_ref, e1_ref, r0_ref, r1_ref, cnt_out_ref,
               cnt_ref):
    i = pl.program_id(0)
    logits = jnp.dot(x_ref[...], wgt_ref[...],
                     preferred_element_type=jnp.float32) + bg_ref[...]
    m = jnp.max(logits, axis=-1, keepdims=True)
    ex = jnp.exp(logits - m)
    p = ex / jnp.sum(ex, axis=-1, keepdims=True)

    iota = lax.broadcasted_iota(jnp.int32, (TG, E), 1)
    top1 = jnp.max(p, axis=-1, keepdims=True)
    a1 = jnp.min(jnp.where(p == top1, iota, E), axis=-1, keepdims=True)
    m1 = iota == a1
    p2 = jnp.where(m1, -jnp.inf, p)
    top2 = jnp.max(p2, axis=-1, keepdims=True)
    a2 = jnp.min(jnp.where(p2 == top2, iota, E), axis=-1, keepdims=True)
    m2 = iota == a2
    maskf = (m1 | m2).astype(jnp.float32)

    @pl.when(i == 0)
    def _():
        cnt_ref[...] = jnp.zeros_like(cnt_ref)

    ri = lax.broadcasted_iota(jnp.int32, (TG, TG), 0)
    ci = lax.broadcasted_iota(jnp.int32, (TG, TG), 1)
    lstrict = (ci < ri).astype(jnp.float32)
    ranks_in = jnp.dot(lstrict, maskf, preferred_element_type=jnp.float32)
    ranks = cnt_ref[...] + ranks_in.astype(jnp.int32)

    r0_ref[...] = jnp.sum(jnp.where(iota == a1, ranks, 0),
                          axis=1, keepdims=True)
    r1_ref[...] = jnp.sum(jnp.where(iota == a2, ranks, 0),
                          axis=1, keepdims=True)
    new_cnt = cnt_ref[...] + jnp.sum(maskf, axis=0,
                                     keepdims=True).astype(jnp.int32)
    cnt_ref[...] = new_cnt
    cnt_out_ref[...] = new_cnt

    s128_ref[...] = jnp.broadcast_to(top1 + top2, (TG, 128))
    e0_ref[...] = a1
    e1_ref[...] = a2


def _gate(x, Wg, bg):
    return pl.pallas_call(
        _gate_body,
        grid=(N // TG,),
        in_specs=[
            pl.BlockSpec((TG, D), lambda i: (i, 0)),
            pl.BlockSpec((D, E), lambda i: (0, 0)),
            pl.BlockSpec((1, E), lambda i: (0, 0)),
        ],
        out_specs=[
            pl.BlockSpec((TG, 128), lambda i: (i, 0)),
            pl.BlockSpec((TG, 1), lambda i: (i, 0)),
            pl.BlockSpec((TG, 1), lambda i: (i, 0)),
            pl.BlockSpec((TG, 1), lambda i: (i, 0)),
            pl.BlockSpec((TG, 1), lambda i: (i, 0)),
            pl.BlockSpec((1, E), lambda i: (0, 0)),
        ],
        out_shape=[
            jax.ShapeDtypeStruct((N, 128), jnp.float32),
            jax.ShapeDtypeStruct((N, 1), jnp.int32),
            jax.ShapeDtypeStruct((N, 1), jnp.int32),
            jax.ShapeDtypeStruct((N, 1), jnp.int32),
            jax.ShapeDtypeStruct((N, 1), jnp.int32),
            jax.ShapeDtypeStruct((1, E), jnp.int32),
        ],
        scratch_shapes=[pltpu.VMEM((1, E), jnp.int32)],
    )(x, Wg.T, bg.reshape(1, E))


_TRIL = None


def _routing(counts, e0, e1, r0, r1):
    counts = counts[0]
    padded = ((counts + TM - 1) // TM) * TM
    tril = (lax.broadcasted_iota(jnp.int32, (E, E), 1)
            <= lax.broadcasted_iota(jnp.int32, (E, E), 0)).astype(jnp.int32)
    cpad = tril @ padded
    poffs = cpad - padded

    e0f, e1f = e0[:, 0], e1[:, 0]
    pos0 = (poffs[e0f] + r0[:, 0]).astype(jnp.int32)
    pos1 = (poffs[e1f] + r1[:, 0]).astype(jnp.int32)

    tok = jnp.arange(N, dtype=jnp.int32)
    pos = jnp.concatenate([pos0, pos1])
    sorted_ids = jnp.zeros((TOT,), jnp.int32).at[pos].set(
        jnp.concatenate([tok, tok]))

    tile_start = jnp.arange(NT, dtype=jnp.int32) * TM
    te = jnp.minimum(
        jnp.sum((tile_start[:, None] >= cpad[None, :]).astype(jnp.int32),
                axis=1), E - 1).astype(jnp.int32)

    bnd = (te[1:] != te[:-1]).astype(jnp.int32)
    tri = (lax.broadcasted_iota(jnp.int32, (NT - 1, NT - 1), 1)
           <= lax.broadcasted_iota(jnp.int32, (NT - 1, NT - 1), 0)
           ).astype(jnp.int32)
    seg = jnp.concatenate([jnp.zeros((1,), jnp.int32), tri @ bnd])
    segexp = jnp.full((NT,), te[-1], jnp.int32).at[seg].set(te)
    nxte = segexp[jnp.minimum(seg + 1, NT - 1)].astype(jnp.int32)
    return sorted_ids, pos0, pos1, te, seg, nxte


GNB = 2
GH = G_PER_W // 2

def _sc_gather(x, sorted_ids):
    mesh = plsc.VectorSubcoreMesh(core_axis_name="c", subcore_axis_name="s")

    @functools.partial(
        pl.kernel,
        mesh=mesh,
        out_type=jax.ShapeDtypeStruct((TOT, D), jnp.float32),
        scratch_types=[pltpu.VMEM((G_PER_W,), jnp.int32)]
        + [pltpu.VMEM((GCH, D), jnp.float32)] * (2 * GNB)
        + [pltpu.SemaphoreType.DMA] * (4 * GNB),
    )
    def k(x_hbm, ids_hbm, xg_hbm, idx_v, *rest):
        bufs = rest[:2 * GNB]
        gsem = rest[2 * GNB:4 * GNB]
        osem = rest[4 * GNB:6 * GNB]
        wid = lax.axis_index("s") * NC + lax.axis_index("c")
        base = wid * G_PER_W
        pltpu.sync_copy(ids_hbm.at[pl.ds(base, G_PER_W)], idx_v)
        nch = GH // GCH
        gh = [None] * (2 * GNB)
        oh = [None] * (2 * GNB)
        for c in range(nch + GNB - 1):
            for s in range(2):
                if c < nch:
                    b = s * GNB + (c % GNB)
                    if c >= GNB:
                        oh[b].wait()
                    off = s * GH + c * GCH
                    gh[b] = pltpu.async_copy(
                        x_hbm.at[idx_v.at[pl.ds(off, GCH)]], bufs[b], gsem[b])
            d = c - (GNB - 1)
            if d >= 0:
                for s in range(2):
                    pb = s * GNB + (d % GNB)
                    off = s * GH + d * GCH
                    gh[pb].wait()
                    oh[pb] = pltpu.async_copy(
                        bufs[pb], xg_hbm.at[pl.ds(base + off, GCH)], osem[pb])
        for s in range(2):
            for k_ in range(GNB):
                oh[s * GNB + (nch - GNB + k_) % GNB].wait()

    return k(x, sorted_ids)


def _gmm_body(te_ref, seg_ref, nxte_ref, xg_ref, w_hbm, b_ref, y_ref,
              wb0, wb1, wb2, sw0, sw1, sw2):
    i = pl.program_id(0)
    wbufs = (wb0, wb1, wb2)
    sems = (sw0, sw1, sw2)
    seg = seg_ref[i]
    slot = lax.rem(seg, 3)
    nslot = lax.rem(seg + 1, 3)
    prev_seg = seg_ref[lax.max(i - 1, 0)]
    first = (i == 0) | (seg != prev_seg)

    @pl.when(i == 0)
    def _():
        for s in range(3):
            @pl.when(slot == s)
            def _():
                pltpu.make_async_copy(
                    w_hbm.at[te_ref[0]], wbufs[s], sems[s]).start()

    @pl.when(first)
    def _():
        for s in range(3):
            @pl.when(nslot == s)
            def _():
                pltpu.make_async_copy(
                    w_hbm.at[nxte_ref[i]], wbufs[s], sems[s]).start()
        for s in range(3):
            @pl.when(slot == s)
            def _():
                pltpu.make_async_copy(
                    w_hbm.at[te_ref[i]], wbufs[s], sems[s]).wait()

    for s in range(3):
        @pl.when(slot == s)
        def _():
            acc = lax.dot_general(
                xg_ref[...].astype(jnp.bfloat16),
                wbufs[s][...].astype(jnp.bfloat16),
                (((1,), (1,)), ((), ())),
                preferred_element_type=jnp.float32,
            )
            y_ref[...] = acc + b_ref[0]

    @pl.when(i == NT - 1)
    def _():
        for s in range(3):
            @pl.when(nslot == s)
            def _():
                pltpu.make_async_copy(
                    w_hbm.at[nxte_ref[i]], wbufs[s], sems[s]).wait()


def _gmm(xg, W, b, te, seg, nxte):
    grid_spec = pltpu.PrefetchScalarGridSpec(
        num_scalar_prefetch=3,
        grid=(NT,),
        in_specs=[
            pl.BlockSpec((TM, D), lambda i, te, seg, nxte: (i, 0)),
            pl.BlockSpec(memory_space=pl.ANY),
            pl.BlockSpec((1, 1, D), lambda i, te, seg, nxte: (te[i], 0, 0)),
        ],
        out_specs=pl.BlockSpec((TM, D), lambda i, te, seg, nxte: (i, 0)),
        scratch_shapes=[
            pltpu.VMEM((D, D), jnp.float32),
            pltpu.VMEM((D, D), jnp.float32),
            pltpu.VMEM((D, D), jnp.float32),
            pltpu.SemaphoreType.DMA,
            pltpu.SemaphoreType.DMA,
            pltpu.SemaphoreType.DMA,
        ],
    )
    return pl.pallas_call(
        _gmm_body,
        grid_spec=grid_spec,
        out_shape=jax.ShapeDtypeStruct((TOT, D), jnp.float32),
    )(te, seg, nxte, xg, W, b.reshape(E, 1, D))


def _sc_combine(y, s128, pos0, pos1):
    mesh = plsc.VectorSubcoreMesh(core_axis_name="c", subcore_axis_name="s")

    @functools.partial(
        pl.kernel,
        mesh=mesh,
        out_type=jax.ShapeDtypeStruct((N, D), jnp.float32),
        scratch_types=[
            pltpu.VMEM((C_PER_W,), jnp.int32),
            pltpu.VMEM((C_PER_W,), jnp.int32),
            pltpu.VMEM((CCH, D), jnp.float32),
            pltpu.VMEM((CCH, D), jnp.float32),
            pltpu.VMEM((CCH, D), jnp.float32),
            pltpu.VMEM((CCH, D), jnp.float32),
            pltpu.VMEM((CCH, 128), jnp.float32),
            pltpu.VMEM((CCH, 128), jnp.float32),
            pltpu.SemaphoreType.DMA,
            pltpu.SemaphoreType.DMA,
            pltpu.SemaphoreType.DMA,
            pltpu.SemaphoreType.DMA,
            pltpu.SemaphoreType.DMA,
            pltpu.SemaphoreType.DMA,
            pltpu.SemaphoreType.DMA,
            pltpu.SemaphoreType.DMA,
        ],
    )
    def k(y_hbm, s_hbm, p0_hbm, p1_hbm, out_hbm,
          p0_v, p1_v, a0, a1, b0, b1, s0, s1,
          ga0, ga1, gb0, gb1, gs0, gs1, o0, o1):
        wid = lax.axis_index("s") * NC + lax.axis_index("c")
        base = wid * C_PER_W
        pltpu.sync_copy(p0_hbm.at[pl.ds(base, C_PER_W)], p0_v)
        pltpu.sync_copy(p1_hbm.at[pl.ds(base, C_PER_W)], p1_v)
        ya, yb, sb = (a0, a1), (b0, b1), (s0, s1)
        gasem, gbsem, gssem, osem = (ga0, ga1), (gb0, gb1), (gs0, gs1), (o0, o1)
        ha = [None, None]
        hb = [None, None]
        hs = [None, None]
        oh = [None, None]
        for c in range(CN + 1):
            b = c & 1
            if c < CN:
                if c >= 2:
                    oh[b].wait()
                ha[b] = pltpu.async_copy(
                    y_hbm.at[p0_v.at[pl.ds(c * CCH, CCH)]], ya[b], gasem[b])
                hb[b] = pltpu.async_copy(
                    y_hbm.at[p1_v.at[pl.ds(c * CCH, CCH)]], yb[b], gbsem[b])
                hs[b] = pltpu.async_copy(
                    s_hbm.at[pl.ds(base + c * CCH, CCH)], sb[b], gssem[b])
            if c >= 1:
                pb = (c - 1) & 1
                ha[pb].wait()
                hb[pb].wait()
                hs[pb].wait()
                svecs = [sb[pb][r, pl.ds(0, 16)] for r in range(CCH)]

                def body(j, _, pb=pb, svecs=svecs):
                    sl = pl.ds(j * 16, 16)
                    for r in range(CCH):
                        ya[pb][r, sl] = (ya[pb][r, sl] + yb[pb][r, sl]) * svecs[r]
                    return 0

                lax.fori_loop(0, D // 16, body, 0)
                oh[pb] = pltpu.async_copy(
                    ya[pb], out_hbm.at[pl.ds(base + (c - 1) * CCH, CCH)],
                    osem[pb])
        oh[0].wait()
        oh[1].wait()

    return k(y, s128, pos0, pos1)


def kernel(x, Wg, bg, W, b):
    s128, e0, e1, r0, r1, counts = _gate(x, Wg, bg)
    sorted_ids, pos0, pos1, te, seg, nxte = _routing(counts, e0, e1, r0, r1)
    xg = _sc_gather(x, sorted_ids)
    y = _gmm(xg, W, b, te, seg, nxte)
    return _sc_combine(y, s128, pos0, pos1)


# speedup vs baseline: 1.0031x; 1.0031x over previous
import functools

import jax
import jax.numpy as jnp
from jax import lax
from jax.experimental import pallas as pl
from jax.experimental.pallas import tpu as pltpu
from jax.experimental.pallas import tpu_sc as plsc

E = 8
D = 2048
N = 2048
TM = 128
TN = 1024
TOT = N * 2 + E * TM
NT = TOT // TM

NC = 2
NS = 16
NW = NC * NS

GCH = 16
G_PER_W = TOT // NW
GN = G_PER_W // GCH
CCH = 8
C_PER_W = N // NW
CN = C_PER_W // CCH


TG = 256


def _gate_body(x_ref, wgt_ref, bg_ref,
               s128_ref, e0_ref, e1_ref, r0_ref, r1_ref, cnt_out_ref,
               cnt_ref):
    i = pl.program_id(0)
    logits = jnp.dot(x_ref[...], wgt_ref[...],
                     preferred_element_type=jnp.float32) + bg_ref[...]
    m = jnp.max(logits, axis=-1, keepdims=True)
    ex = jnp.exp(logits - m)
    p = ex / jnp.sum(ex, axis=-1, keepdims=True)

    iota = lax.broadcasted_iota(jnp.int32, (TG, E), 1)
    top1 = jnp.max(p, axis=-1, keepdims=True)
    a1 = jnp.min(jnp.where(p == top1, iota, E), axis=-1, keepdims=True)
    m1 = iota == a1
    p2 = jnp.where(m1, -jnp.inf, p)
    top2 = jnp.max(p2, axis=-1, keepdims=True)
    a2 = jnp.min(jnp.where(p2 == top2, iota, E), axis=-1, keepdims=True)
    m2 = iota == a2
    maskf = (m1 | m2).astype(jnp.float32)

    @pl.when(i == 0)
    def _():
        cnt_ref[...] = jnp.zeros_like(cnt_ref)

    ri = lax.broadcasted_iota(jnp.int32, (TG, TG), 0)
    ci = lax.broadcasted_iota(jnp.int32, (TG, TG), 1)
    lstrict = (ci < ri).astype(jnp.float32)
    ranks_in = jnp.dot(lstrict, maskf, preferred_element_type=jnp.float32)
    ranks = cnt_ref[...] + ranks_in.astype(jnp.int32)

    r0_ref[...] = jnp.sum(jnp.where(iota == a1, ranks, 0),
                          axis=1, keepdims=True)
    r1_ref[...] = jnp.sum(jnp.where(iota == a2, ranks, 0),
                          axis=1, keepdims=True)
    new_cnt = cnt_ref[...] + jnp.sum(maskf, axis=0,
                                     keepdims=True).astype(jnp.int32)
    cnt_ref[...] = new_cnt
    cnt_out_ref[...] = new_cnt

    s128_ref[...] = jnp.broadcast_to(top1 + top2, (TG, 128))
    e0_ref[...] = a1
    e1_ref[...] = a2


def _gate(x, Wg, bg):
    return pl.pallas_call(
        _gate_body,
        grid=(N // TG,),
        in_specs=[
            pl.BlockSpec((TG, D), lambda i: (i, 0)),
            pl.BlockSpec((D, E), lambda i: (0, 0)),
            pl.BlockSpec((1, E), lambda i: (0, 0)),
        ],
        out_specs=[
            pl.BlockSpec((TG, 128), lambda i: (i, 0)),
            pl.BlockSpec((TG, 1), lambda i: (i, 0)),
            pl.BlockSpec((TG, 1), lambda i: (i, 0)),
            pl.BlockSpec((TG, 1), lambda i: (i, 0)),
            pl.BlockSpec((TG, 1), lambda i: (i, 0)),
            pl.BlockSpec((1, E), lambda i: (0, 0)),
        ],
        out_shape=[
            jax.ShapeDtypeStruct((N, 128), jnp.float32),
            jax.ShapeDtypeStruct((N, 1), jnp.int32),
            jax.ShapeDtypeStruct((N, 1), jnp.int32),
            jax.ShapeDtypeStruct((N, 1), jnp.int32),
            jax.ShapeDtypeStruct((N, 1), jnp.int32),
            jax.ShapeDtypeStruct((1, E), jnp.int32),
        ],
        scratch_shapes=[pltpu.VMEM((1, E), jnp.int32)],
    )(x, Wg.T, bg.reshape(1, E))


_TRIL = None


def _routing(counts, e0, e1, r0, r1):
    counts = counts[0]
    padded = ((counts + TM - 1) // TM) * TM
    tril = (lax.broadcasted_iota(jnp.int32, (E, E), 1)
            <= lax.broadcasted_iota(jnp.int32, (E, E), 0)).astype(jnp.int32)
    cpad = tril @ padded
    poffs = cpad - padded

    e0f, e1f = e0[:, 0], e1[:, 0]
    pos0 = (poffs[e0f] + r0[:, 0]).astype(jnp.int32)
    pos1 = (poffs[e1f] + r1[:, 0]).astype(jnp.int32)

    tok = jnp.arange(N, dtype=jnp.int32)
    pos = jnp.concatenate([pos0, pos1])
    sorted_ids = jnp.zeros((TOT,), jnp.int32).at[pos].set(
        jnp.concatenate([tok, tok]))

    tile_start = jnp.arange(NT, dtype=jnp.int32) * TM
    te = jnp.minimum(
        jnp.sum((tile_start[:, None] >= cpad[None, :]).astype(jnp.int32),
                axis=1), E - 1).astype(jnp.int32)

    bnd = (te[1:] != te[:-1]).astype(jnp.int32)
    tri = (lax.broadcasted_iota(jnp.int32, (NT - 1, NT - 1), 1)
           <= lax.broadcasted_iota(jnp.int32, (NT - 1, NT - 1), 0)
           ).astype(jnp.int32)
    seg = jnp.concatenate([jnp.zeros((1,), jnp.int32), tri @ bnd])
    segexp = jnp.full((NT,), te[-1], jnp.int32).at[seg].set(te)
    nxte = segexp[jnp.minimum(seg + 1, NT - 1)].astype(jnp.int32)
    return sorted_ids, pos0, pos1, te, seg, nxte


GNB = 3

def _sc_gather(x, sorted_ids):
    mesh = plsc.VectorSubcoreMesh(core_axis_name="c", subcore_axis_name="s")

    @functools.partial(
        pl.kernel,
        mesh=mesh,
        out_type=jax.ShapeDtypeStruct((TOT, D), jnp.float32),
        scratch_types=[pltpu.VMEM((G_PER_W,), jnp.int32)]
        + [pltpu.VMEM((GCH, D), jnp.float32)] * GNB
        + [pltpu.SemaphoreType.DMA] * (2 * GNB),
    )
    def k(x_hbm, ids_hbm, xg_hbm, idx_v, *rest):
        bufs = rest[:GNB]
        gsem = rest[GNB:2 * GNB]
        osem = rest[2 * GNB:3 * GNB]
        wid = lax.axis_index("s") * NC + lax.axis_index("c")
        base = wid * G_PER_W
        pltpu.sync_copy(ids_hbm.at[pl.ds(base, G_PER_W)], idx_v)
        gh = [None] * GNB
        oh = [None] * GNB
        for c in range(GN + GNB - 1):
            if c < GN:
                b = c % GNB
                if c >= GNB:
                    oh[b].wait()
                gh[b] = pltpu.async_copy(
                    x_hbm.at[idx_v.at[pl.ds(c * GCH, GCH)]], bufs[b], gsem[b])
            d = c - (GNB - 1)
            if d >= 0:
                pb = d % GNB
                gh[pb].wait()
                oh[pb] = pltpu.async_copy(
                    bufs[pb], xg_hbm.at[pl.ds(base + d * GCH, GCH)], osem[pb])
        for k_ in range(GNB):
            oh[(GN - GNB + k_) % GNB].wait()

    return k(x, sorted_ids)


def _gmm_body(te_ref, seg_ref, nxte_ref, xg_ref, w_hbm, b_ref, y_ref,
              wb0, wb1, wb2, sw0, sw1, sw2):
    i = pl.program_id(0)
    wbufs = (wb0, wb1, wb2)
    sems = (sw0, sw1, sw2)
    seg = seg_ref[i]
    slot = lax.rem(seg, 3)
    nslot = lax.rem(seg + 1, 3)
    prev_seg = seg_ref[lax.max(i - 1, 0)]
    first = (i == 0) | (seg != prev_seg)

    @pl.when(i == 0)
    def _():
        for s in range(3):
            @pl.when(slot == s)
            def _():
                pltpu.make_async_copy(
                    w_hbm.at[te_ref[0]], wbufs[s], sems[s]).start()

    @pl.when(first)
    def _():
        for s in range(3):
            @pl.when(nslot == s)
            def _():
                pltpu.make_async_copy(
                    w_hbm.at[nxte_ref[i]], wbufs[s], sems[s]).start()
        for s in range(3):
            @pl.when(slot == s)
            def _():
                pltpu.make_async_copy(
                    w_hbm.at[te_ref[i]], wbufs[s], sems[s]).wait()

    for s in range(3):
        @pl.when(slot == s)
        def _():
            acc = lax.dot_general(
                xg_ref[...].astype(jnp.bfloat16),
                wbufs[s][...].astype(jnp.bfloat16),
                (((1,), (1,)), ((), ())),
                preferred_element_type=jnp.float32,
            )
            y_ref[...] = acc + b_ref[0]

    @pl.when(i == NT - 1)
    def _():
        for s in range(3):
            @pl.when(nslot == s)
            def _():
                pltpu.make_async_copy(
                    w_hbm.at[nxte_ref[i]], wbufs[s], sems[s]).wait()


def _gmm(xg, W, b, te, seg, nxte):
    grid_spec = pltpu.PrefetchScalarGridSpec(
        num_scalar_prefetch=3,
        grid=(NT,),
        in_specs=[
            pl.BlockSpec((TM, D), lambda i, te, seg, nxte: (i, 0)),
            pl.BlockSpec(memory_space=pl.ANY),
            pl.BlockSpec((1, 1, D), lambda i, te, seg, nxte: (te[i], 0, 0)),
        ],
        out_specs=pl.BlockSpec((TM, D), lambda i, te, seg, nxte: (i, 0)),
        scratch_shapes=[
            pltpu.VMEM((D, D), jnp.float32),
            pltpu.VMEM((D, D), jnp.float32),
            pltpu.VMEM((D, D), jnp.float32),
            pltpu.SemaphoreType.DMA,
            pltpu.SemaphoreType.DMA,
            pltpu.SemaphoreType.DMA,
        ],
    )
    return pl.pallas_call(
        _gmm_body,
        grid_spec=grid_spec,
        out_shape=jax.ShapeDtypeStruct((TOT, D), jnp.float32),
    )(te, seg, nxte, xg, W, b.reshape(E, 1, D))


def _sc_combine(y, s128, pos0, pos1):
    mesh = plsc.VectorSubcoreMesh(core_axis_name="c", subcore_axis_name="s")

    @functools.partial(
        pl.kernel,
        mesh=mesh,
        out_type=jax.ShapeDtypeStruct((N, D), jnp.float32),
        scratch_types=[
            pltpu.VMEM((C_PER_W,), jnp.int32),
            pltpu.VMEM((C_PER_W,), jnp.int32),
            pltpu.VMEM((CCH, D), jnp.float32),
            pltpu.VMEM((CCH, D), jnp.float32),
            pltpu.VMEM((CCH, D), jnp.float32),
            pltpu.VMEM((CCH, D), jnp.float32),
            pltpu.VMEM((CCH, 128), jnp.float32),
            pltpu.VMEM((CCH, 128), jnp.float32),
            pltpu.SemaphoreType.DMA,
            pltpu.SemaphoreType.DMA,
            pltpu.SemaphoreType.DMA,
            pltpu.SemaphoreType.DMA,
            pltpu.SemaphoreType.DMA,
            pltpu.SemaphoreType.DMA,
            pltpu.SemaphoreType.DMA,
            pltpu.SemaphoreType.DMA,
        ],
    )
    def k(y_hbm, s_hbm, p0_hbm, p1_hbm, out_hbm,
          p0_v, p1_v, a0, a1, b0, b1, s0, s1,
          ga0, ga1, gb0, gb1, gs0, gs1, o0, o1):
        wid = lax.axis_index("s") * NC + lax.axis_index("c")
        base = wid * C_PER_W
        pltpu.sync_copy(p0_hbm.at[pl.ds(base, C_PER_W)], p0_v)
        pltpu.sync_copy(p1_hbm.at[pl.ds(base, C_PER_W)], p1_v)
        ya, yb, sb = (a0, a1), (b0, b1), (s0, s1)
        gasem, gbsem, gssem, osem = (ga0, ga1), (gb0, gb1), (gs0, gs1), (o0, o1)
        ha = [None, None]
        hb = [None, None]
        hs = [None, None]
        oh = [None, None]
        for c in range(CN + 1):
            b = c & 1
            if c < CN:
                if c >= 2:
                    oh[b].wait()
                ha[b] = pltpu.async_copy(
                    y_hbm.at[p0_v.at[pl.ds(c * CCH, CCH)]], ya[b], gasem[b])
                hb[b] = pltpu.async_copy(
                    y_hbm.at[p1_v.at[pl.ds(c * CCH, CCH)]], yb[b], gbsem[b])
                hs[b] = pltpu.async_copy(
                    s_hbm.at[pl.ds(base + c * CCH, CCH)], sb[b], gssem[b])
            if c >= 1:
                pb = (c - 1) & 1
                ha[pb].wait()
                hb[pb].wait()
                hs[pb].wait()
                svecs = [sb[pb][r, pl.ds(0, 16)] for r in range(CCH)]

                def body(j, _, pb=pb, svecs=svecs):
                    sl = pl.ds(j * 16, 16)
                    for r in range(CCH):
                        ya[pb][r, sl] = (ya[pb][r, sl] + yb[pb][r, sl]) * svecs[r]
                    return 0

                lax.fori_loop(0, D // 16, body, 0)
                oh[pb] = pltpu.async_copy(
                    ya[pb], out_hbm.at[pl.ds(base + (c - 1) * CCH, CCH)],
                    osem[pb])
        oh[0].wait()
        oh[1].wait()

    return k(y, s128, pos0, pos1)


def kernel(x, Wg, bg, W, b):
    s128, e0, e1, r0, r1, counts = _gate(x, Wg, bg)
    sorted_ids, pos0, pos1, te, seg, nxte = _routing(counts, e0, e1, r0, r1)
    xg = _sc_gather(x, sorted_ids)
    y = _gmm(xg, W, b, te, seg, nxte)
    return _sc_combine(y, s128, pos0, pos1)


# speedup vs baseline: 1.0047x; 1.0016x over previous
import functools

import jax
import jax.numpy as jnp
from jax import lax
from jax.experimental import pallas as pl
from jax.experimental.pallas import tpu as pltpu
from jax.experimental.pallas import tpu_sc as plsc

E = 8
D = 2048
N = 2048
TM = 128
TOT = N * 2 + E * TM
NT = TOT // TM

NC = 2
NS = 16
NW = NC * NS

GCH = 16
G_PER_W = TOT // NW
GN = G_PER_W // GCH
CCH = 8
C_PER_W = N // NW
CN = C_PER_W // CCH


TG = 256


def _gate_body(x_ref, wgt_ref, bg_ref,
               s128_ref, e0_ref, e1_ref, r0_ref, r1_ref, cnt_out_ref,
               cnt_ref):
    i = pl.program_id(0)
    logits = jnp.dot(x_ref[...], wgt_ref[...],
                     preferred_element_type=jnp.float32) + bg_ref[...]
    m = jnp.max(logits, axis=-1, keepdims=True)
    ex = jnp.exp(logits - m)
    p = ex / jnp.sum(ex, axis=-1, keepdims=True)

    iota = lax.broadcasted_iota(jnp.int32, (TG, E), 1)
    top1 = jnp.max(p, axis=-1, keepdims=True)
    a1 = jnp.min(jnp.where(p == top1, iota, E), axis=-1, keepdims=True)
    m1 = iota == a1
    p2 = jnp.where(m1, -jnp.inf, p)
    top2 = jnp.max(p2, axis=-1, keepdims=True)
    a2 = jnp.min(jnp.where(p2 == top2, iota, E), axis=-1, keepdims=True)
    m2 = iota == a2
    maskf = (m1 | m2).astype(jnp.float32)

    @pl.when(i == 0)
    def _():
        cnt_ref[...] = jnp.zeros_like(cnt_ref)

    ri = lax.broadcasted_iota(jnp.int32, (TG, TG), 0)
    ci = lax.broadcasted_iota(jnp.int32, (TG, TG), 1)
    lstrict = (ci < ri).astype(jnp.float32)
    ranks_in = jnp.dot(lstrict, maskf, preferred_element_type=jnp.float32)
    ranks = cnt_ref[...] + ranks_in.astype(jnp.int32)

    r0_ref[...] = jnp.sum(jnp.where(iota == a1, ranks, 0),
                          axis=1, keepdims=True)
    r1_ref[...] = jnp.sum(jnp.where(iota == a2, ranks, 0),
                          axis=1, keepdims=True)
    new_cnt = cnt_ref[...] + jnp.sum(maskf, axis=0,
                                     keepdims=True).astype(jnp.int32)
    cnt_ref[...] = new_cnt
    cnt_out_ref[...] = new_cnt

    s128_ref[...] = jnp.broadcast_to(top1 + top2, (TG, 128))
    e0_ref[...] = a1
    e1_ref[...] = a2


def _gate(x, Wg, bg):
    return pl.pallas_call(
        _gate_body,
        grid=(N // TG,),
        in_specs=[
            pl.BlockSpec((TG, D), lambda i: (i, 0)),
            pl.BlockSpec((D, E), lambda i: (0, 0)),
            pl.BlockSpec((1, E), lambda i: (0, 0)),
        ],
        out_specs=[
            pl.BlockSpec((TG, 128), lambda i: (i, 0)),
            pl.BlockSpec((TG, 1), lambda i: (i, 0)),
            pl.BlockSpec((TG, 1), lambda i: (i, 0)),
            pl.BlockSpec((TG, 1), lambda i: (i, 0)),
            pl.BlockSpec((TG, 1), lambda i: (i, 0)),
            pl.BlockSpec((1, E), lambda i: (0, 0)),
        ],
        out_shape=[
            jax.ShapeDtypeStruct((N, 128), jnp.float32),
            jax.ShapeDtypeStruct((N, 1), jnp.int32),
            jax.ShapeDtypeStruct((N, 1), jnp.int32),
            jax.ShapeDtypeStruct((N, 1), jnp.int32),
            jax.ShapeDtypeStruct((N, 1), jnp.int32),
            jax.ShapeDtypeStruct((1, E), jnp.int32),
        ],
        scratch_shapes=[pltpu.VMEM((1, E), jnp.int32)],
    )(x, Wg.T, bg.reshape(1, E))


def _routing(counts, e0, e1, r0, r1):
    counts = counts[0]
    padded = ((counts + TM - 1) // TM) * TM
    tril = (lax.broadcasted_iota(jnp.int32, (E, E), 1)
            <= lax.broadcasted_iota(jnp.int32, (E, E), 0)).astype(jnp.int32)
    cpad = tril @ padded
    poffs = cpad - padded

    e0f, e1f = e0[:, 0], e1[:, 0]
    pos0 = (poffs[e0f] + r0[:, 0]).astype(jnp.int32)
    pos1 = (poffs[e1f] + r1[:, 0]).astype(jnp.int32)

    tok = jnp.arange(N, dtype=jnp.int32)
    pos = jnp.concatenate([pos0, pos1])
    sorted_ids = jnp.zeros((TOT,), jnp.int32).at[pos].set(
        jnp.concatenate([tok, tok]))

    tile_start = jnp.arange(NT, dtype=jnp.int32) * TM
    te = jnp.minimum(
        jnp.sum((tile_start[:, None] >= cpad[None, :]).astype(jnp.int32),
                axis=1), E - 1).astype(jnp.int32)

    bnd = (te[1:] != te[:-1]).astype(jnp.int32)
    tri = (lax.broadcasted_iota(jnp.int32, (NT - 1, NT - 1), 1)
           <= lax.broadcasted_iota(jnp.int32, (NT - 1, NT - 1), 0)
           ).astype(jnp.int32)
    seg = jnp.concatenate([jnp.zeros((1,), jnp.int32), tri @ bnd])
    segexp = jnp.full((NT,), te[-1], jnp.int32).at[seg].set(te)
    nxte = segexp[jnp.minimum(seg + 1, NT - 1)].astype(jnp.int32)
    return sorted_ids, pos0, pos1, te, seg, nxte


GNB = 3

def _sc_gather(x, sorted_ids):
    mesh = plsc.VectorSubcoreMesh(core_axis_name="c", subcore_axis_name="s")

    @functools.partial(
        pl.kernel,
        mesh=mesh,
        out_type=jax.ShapeDtypeStruct((TOT, D), jnp.float32),
        scratch_types=[pltpu.VMEM((G_PER_W,), jnp.int32)]
        + [pltpu.VMEM((GCH, D), jnp.float32)] * GNB
        + [pltpu.SemaphoreType.DMA] * (2 * GNB),
    )
    def k(x_hbm, ids_hbm, xg_hbm, idx_v, *rest):
        bufs = rest[:GNB]
        gsem = rest[GNB:2 * GNB]
        osem = rest[2 * GNB:3 * GNB]
        wid = lax.axis_index("s") * NC + lax.axis_index("c")
        base = wid * G_PER_W
        pltpu.sync_copy(ids_hbm.at[pl.ds(base, G_PER_W)], idx_v)
        gh = [None] * GNB
        oh = [None] * GNB
        for c in range(GN + GNB - 1):
            if c < GN:
                b = c % GNB
                if c >= GNB:
                    oh[b].wait()
                gh[b] = pltpu.async_copy(
                    x_hbm.at[idx_v.at[pl.ds(c * GCH, GCH)]], bufs[b], gsem[b])
            d = c - (GNB - 1)
            if d >= 0:
                pb = d % GNB
                gh[pb].wait()
                oh[pb] = pltpu.async_copy(
                    bufs[pb], xg_hbm.at[pl.ds(base + d * GCH, GCH)], osem[pb])
        for k_ in range(GNB):
            oh[(GN - GNB + k_) % GNB].wait()

    return k(x, sorted_ids)


def _gmm_body(te_ref, seg_ref, nxte_ref, xg_ref, w_hbm, b_ref, y_ref,
              wb0, wb1, sw0, sw1):
    i = pl.program_id(0)
    wbufs = (wb0, wb1)
    sems = (sw0, sw1)
    seg = seg_ref[i]
    slot = lax.rem(seg, 2)
    nslot = lax.rem(seg + 1, 2)
    prev_seg = seg_ref[lax.max(i - 1, 0)]
    first = (i == 0) | (seg != prev_seg)

    @pl.when(i == 0)
    def _():
        for s in range(2):
            @pl.when(slot == s)
            def _():
                pltpu.make_async_copy(
                    w_hbm.at[te_ref[0]], wbufs[s], sems[s]).start()

    @pl.when(first)
    def _():
        for s in range(2):
            @pl.when(nslot == s)
            def _():
                pltpu.make_async_copy(
                    w_hbm.at[nxte_ref[i]], wbufs[s], sems[s]).start()
        for s in range(2):
            @pl.when(slot == s)
            def _():
                pltpu.make_async_copy(
                    w_hbm.at[te_ref[i]], wbufs[s], sems[s]).wait()

    for s in range(2):
        @pl.when(slot == s)
        def _():
            acc = lax.dot_general(
                xg_ref[...].astype(jnp.bfloat16),
                wbufs[s][...].astype(jnp.bfloat16),
                (((1,), (1,)), ((), ())),
                preferred_element_type=jnp.float32,
            )
            y_ref[...] = acc + b_ref[0]

    @pl.when(i == NT - 1)
    def _():
        for s in range(2):
            @pl.when(nslot == s)
            def _():
                pltpu.make_async_copy(
                    w_hbm.at[nxte_ref[i]], wbufs[s], sems[s]).wait()


def _gmm(xg, W, b, te, seg, nxte):
    grid_spec = pltpu.PrefetchScalarGridSpec(
        num_scalar_prefetch=3,
        grid=(NT,),
        in_specs=[
            pl.BlockSpec((TM, D), lambda i, te, seg, nxte: (i, 0)),
            pl.BlockSpec(memory_space=pl.ANY),
            pl.BlockSpec((1, 1, D), lambda i, te, seg, nxte: (te[i], 0, 0)),
        ],
        out_specs=pl.BlockSpec((TM, D), lambda i, te, seg, nxte: (i, 0)),
        scratch_shapes=[
            pltpu.VMEM((D, D), jnp.float32),
            pltpu.VMEM((D, D), jnp.float32),
            pltpu.SemaphoreType.DMA,
            pltpu.SemaphoreType.DMA,
        ],
    )
    return pl.pallas_call(
        _gmm_body,
        grid_spec=grid_spec,
        out_shape=jax.ShapeDtypeStruct((TOT, D), jnp.float32),
    )(te, seg, nxte, xg, W, b.reshape(E, 1, D))


def _sc_combine(y, s128, pos0, pos1):
    mesh = plsc.VectorSubcoreMesh(core_axis_name="c", subcore_axis_name="s")

    @functools.partial(
        pl.kernel,
        mesh=mesh,
        out_type=jax.ShapeDtypeStruct((N, D), jnp.float32),
        scratch_types=[
            pltpu.VMEM((C_PER_W,), jnp.int32),
            pltpu.VMEM((C_PER_W,), jnp.int32),
            pltpu.VMEM((CCH, D), jnp.float32),
            pltpu.VMEM((CCH, D), jnp.float32),
            pltpu.VMEM((CCH, D), jnp.float32),
            pltpu.VMEM((CCH, D), jnp.float32),
            pltpu.VMEM((CCH, 128), jnp.float32),
            pltpu.VMEM((CCH, 128), jnp.float32),
            pltpu.SemaphoreType.DMA,
            pltpu.SemaphoreType.DMA,
            pltpu.SemaphoreType.DMA,
            pltpu.SemaphoreType.DMA,
            pltpu.SemaphoreType.DMA,
            pltpu.SemaphoreType.DMA,
            pltpu.SemaphoreType.DMA,
            pltpu.SemaphoreType.DMA,
        ],
    )
    def k(y_hbm, s_hbm, p0_hbm, p1_hbm, out_hbm,
          p0_v, p1_v, a0, a1, b0, b1, s0, s1,
          ga0, ga1, gb0, gb1, gs0, gs1, o0, o1):
        wid = lax.axis_index("s") * NC + lax.axis_index("c")
        base = wid * C_PER_W
        pltpu.sync_copy(p0_hbm.at[pl.ds(base, C_PER_W)], p0_v)
        pltpu.sync_copy(p1_hbm.at[pl.ds(base, C_PER_W)], p1_v)
        ya, yb, sb = (a0, a1), (b0, b1), (s0, s1)
        gasem, gbsem, gssem, osem = (ga0, ga1), (gb0, gb1), (gs0, gs1), (o0, o1)
        ha = [None, None]
        hb = [None, None]
        hs = [None, None]
        oh = [None, None]
        for c in range(CN + 1):
            b = c & 1
            if c < CN:
                if c >= 2:
                    oh[b].wait()
                ha[b] = pltpu.async_copy(
                    y_hbm.at[p0_v.at[pl.ds(c * CCH, CCH)]], ya[b], gasem[b])
                hb[b] = pltpu.async_copy(
                    y_hbm.at[p1_v.at[pl.ds(c * CCH, CCH)]], yb[b], gbsem[b])
                hs[b] = pltpu.async_copy(
                    s_hbm.at[pl.ds(base + c * CCH, CCH)], sb[b], gssem[b])
            if c >= 1:
                pb = (c - 1) & 1
                ha[pb].wait()
                hb[pb].wait()
                hs[pb].wait()
                svecs = [sb[pb][r, pl.ds(0, 16)] for r in range(CCH)]

                def body(j, _, pb=pb, svecs=svecs):
                    sl = pl.ds(j * 16, 16)
                    for r in range(CCH):
                        ya[pb][r, sl] = (ya[pb][r, sl] + yb[pb][r, sl]) * svecs[r]
                    return 0

                lax.fori_loop(0, D // 16, body, 0)
                oh[pb] = pltpu.async_copy(
                    ya[pb], out_hbm.at[pl.ds(base + (c - 1) * CCH, CCH)],
                    osem[pb])
        oh[0].wait()
        oh[1].wait()

    return k(y, s128, pos0, pos1)


def kernel(x, Wg, bg, W, b):
    s128, e0, e1, r0, r1, counts = _gate(x, Wg, bg)
    sorted_ids, pos0, pos1, te, seg, nxte = _routing(counts, e0, e1, r0, r1)
    xg = _sc_gather(x, sorted_ids)
    y = _gmm(xg, W, b, te, seg, nxte)
    return _sc_combine(y, s128, pos0, pos1)
